# baseline (speedup 1.0000x reference)
import functools

import jax
import jax.numpy as jnp
from jax import lax
from jax.experimental import pallas as pl
from jax.experimental.pallas import tpu as pltpu
from jax.experimental.pallas import tpu_sc as plsc

_N_ENTITY = 100000
_N_RELATION = 500
_D = 32
_D2 = 64
_B = 1024
_REG = 0.05
_T = 2000
_NT = _N_ENTITY // _T
_TM = 25000
_TM_STEPS = _N_ENTITY // _TM

_NC, _NS = 2, 16
_NW = _NC * _NS
_BPW = _B // _NW


@functools.lru_cache(maxsize=1)
def _make_gather3():
    mesh = plsc.VectorSubcoreMesh(core_axis_name="c", subcore_axis_name="s",
                                  num_cores=_NC, num_subcores=_NS)

    @functools.partial(
        pl.kernel,
        mesh=mesh,
        out_type=[jax.ShapeDtypeStruct((_B, _D2), jnp.float32)] * 3,
        scratch_types=[
            [pltpu.VMEM((_BPW,), jnp.int32)] * 3,
            [pltpu.VMEM((_BPW, _D2), jnp.float32)] * 3,
            pltpu.SemaphoreType.DMA,
        ],
    )
    def _gather3(entity_hbm, relation_hbm, anchor_hbm, relind_hbm, ans_hbm,
                 head_out, rel_out, tail_out, idx_vs, rows_vs, sem):
        wid = lax.axis_index("s") * _NC + lax.axis_index("c")
        base = wid * _BPW
        triples = (
            (anchor_hbm, entity_hbm, head_out),
            (relind_hbm, relation_hbm, rel_out),
            (ans_hbm, entity_hbm, tail_out),
        )
        for t, (idx_hbm, table, out) in enumerate(triples):
            pltpu.sync_copy(idx_hbm.at[pl.ds(base, _BPW)], idx_vs[t])
        descs = []
        for t, (idx_hbm, table, out) in enumerate(triples):
            for c in range(_BPW // 16):
                chunk = idx_vs[t][pl.ds(c * 16, 16)]
                for k in range(16):
                    j = c * 16 + k
                    descs.append(pltpu.async_copy(
                        table.at[pl.ds(chunk[k], 1)],
                        rows_vs[t].at[pl.ds(j, 1)], sem))
        for d in descs:
            d.wait()
        for t, (idx_hbm, table, out) in enumerate(triples):
            pltpu.sync_copy(rows_vs[t], out.at[pl.ds(base, _BPW)])

    return _gather3


def _score_body(head_ref, rel_ref, tail_ref, e_ref, out_ref,
                q_ref, m_ref, s_ref, g_ref):
    i = pl.program_id(0)

    @pl.when(i == 0)
    def _init():
        head = head_ref[...]
        rel = rel_ref[...]
        tail = tail_ref[...]
        lhs0, lhs1 = head[:, :_D], head[:, _D:]
        rel0, rel1 = rel[:, :_D], rel[:, _D:]
        q0 = lhs0 * rel0 - lhs1 * rel1
        q1 = lhs0 * rel1 + lhs1 * rel0
        q_ref[...] = jnp.concatenate([q0, q1], axis=1)
        g_ref[...] = jnp.sum(q0 * tail[:, :_D] + q1 * tail[:, _D:],
                             axis=1, keepdims=True)
        m_ref[...] = jnp.full((_B, 1), -jnp.inf, dtype=jnp.float32)
        s_ref[...] = jnp.zeros((_B, 1), dtype=jnp.float32)

    q = q_ref[...]
    e = e_ref[...]
    p = lax.dot_general(q, e, (((1,), (1,)), ((), ())),
                        preferred_element_type=jnp.float32)
    m_old = m_ref[...]
    m_new = jnp.maximum(m_old, jnp.max(p, axis=1, keepdims=True))
    s_ref[...] = (s_ref[...] * jnp.exp(m_old - m_new)
                  + jnp.sum(jnp.exp(p - m_new), axis=1, keepdims=True))
    m_ref[...] = m_new

    @pl.when(i == pl.num_programs(0) - 1)
    def _fin():
        lse = m_ref[...] + jnp.log(s_ref[...])
        l_fit = jnp.mean(lse - g_ref[...])
        head = head_ref[...]
        rel = rel_ref[...]
        tail = tail_ref[...]

        def n3(x):
            a = x[:, :_D] ** 2 + x[:, _D:] ** 2
            return jnp.sum(a * jnp.sqrt(a))

        l_reg = _REG * (n3(head) + n3(rel) + n3(tail)) / _B
        out_ref[0, 0] = l_fit + l_reg


def _score(head, rel, tail, entity_emb):
    return pl.pallas_call(
        _score_body,
        grid=(_NT,),
        in_specs=[
            pl.BlockSpec((_B, _D2), lambda i: (0, 0)),
            pl.BlockSpec((_B, _D2), lambda i: (0, 0)),
            pl.BlockSpec((_B, _D2), lambda i: (0, 0)),
            pl.BlockSpec((_T, _D2), lambda i: (i, 0)),
        ],
        out_specs=pl.BlockSpec(memory_space=pltpu.SMEM),
        out_shape=jax.ShapeDtypeStruct((1, 1), jnp.float32),
        scratch_shapes=[
            pltpu.VMEM((_B, _D2), jnp.float32),
            pltpu.VMEM((_B, 1), jnp.float32),
            pltpu.VMEM((_B, 1), jnp.float32),
            pltpu.VMEM((_B, 1), jnp.float32),
        ],
        compiler_params=pltpu.CompilerParams(
            dimension_semantics=("arbitrary",)),
    )(head, rel, tail, entity_emb)


def _moments_body(e0_ref, e1_ref, s_ref, mom_ref):
    i = pl.program_id(0)

    @pl.when(i == 0)
    def _init():
        s_ref[...] = jnp.zeros((1, _D2), dtype=jnp.float32)
        mom_ref[...] = jnp.zeros((_D2, _D2), dtype=jnp.float32)

    ones = jnp.ones((1, _TM), dtype=jnp.float32)
    for e_ref in (e0_ref, e1_ref):
        e = e_ref[...]
        mom_ref[...] += lax.dot_general(e, e, (((0,), (0,)), ((), ())),
                                        preferred_element_type=jnp.float32)
        s_ref[...] += lax.dot_general(ones, e, (((1,), (0,)), ((), ())),
                                      preferred_element_type=jnp.float32)


def _moments(entity_emb):
    return pl.pallas_call(
        _moments_body,
        grid=(_TM_STEPS // 2,),
        in_specs=[
            pl.BlockSpec((_TM, _D2), lambda i: (i, 0)),
            pl.BlockSpec((_TM, _D2), lambda i: (i + _TM_STEPS // 2, 0)),
        ],
        out_specs=[
            pl.BlockSpec((1, _D2), lambda i: (0, 0)),
            pl.BlockSpec((_D2, _D2), lambda i: (0, 0)),
        ],
        out_shape=[
            jax.ShapeDtypeStruct((1, _D2), jnp.float32),
            jax.ShapeDtypeStruct((_D2, _D2), jnp.float32),
        ],
        compiler_params=pltpu.CompilerParams(
            dimension_semantics=("arbitrary",)),
    )(entity_emb, entity_emb)


def _assemble_body(head_ref, rel_ref, tail_ref, s_ref, mom_ref,
                   out_ref, bnd_ref):
    head = head_ref[...]
    rel = rel_ref[...]
    tail = tail_ref[...]
    lhs0, lhs1 = head[:, :_D], head[:, _D:]
    rel0, rel1 = rel[:, :_D], rel[:, _D:]
    q0 = lhs0 * rel0 - lhs1 * rel1
    q1 = lhs0 * rel1 + lhs1 * rel0
    q = jnp.concatenate([q0, q1], axis=1)
    s1 = jnp.sum(q * s_ref[...], axis=1, keepdims=True)
    qm = lax.dot_general(q, mom_ref[...], (((1,), (0,)), ((), ())),
                         preferred_element_type=jnp.float32)
    s2 = jnp.sum(qm * q, axis=1, keepdims=True)
    lse = jnp.log(jnp.float32(_N_ENTITY) + s1 + 0.5 * s2)
    gold = jnp.sum(q * tail, axis=1, keepdims=True)
    l_fit = jnp.mean(lse - gold)

    def n3(x):
        a = x[:, :_D] ** 2 + x[:, _D:] ** 2
        return jnp.sum(a * jnp.sqrt(a))

    l_reg = _REG * (n3(head) + n3(rel) + n3(tail)) / _B
    out_ref[0, 0] = l_fit + l_reg
    mom = mom_ref[...]
    row = lax.broadcasted_iota(jnp.int32, (_D2, _D2), 0)
    col = lax.broadcasted_iota(jnp.int32, (_D2, _D2), 1)
    tr = jnp.sum(jnp.where(row == col, mom, 0.0))
    maxq2 = jnp.max(jnp.sum(q * q, axis=1, keepdims=True))
    bnd_ref[0, 0] = jnp.sqrt(maxq2 * tr)


def _assemble(head, rel, tail, svec, mom):
    return pl.pallas_call(
        _assemble_body,
        in_specs=[
            pl.BlockSpec((_B, _D2), lambda: (0, 0)),
            pl.BlockSpec((_B, _D2), lambda: (0, 0)),
            pl.BlockSpec((_B, _D2), lambda: (0, 0)),
            pl.BlockSpec((1, _D2), lambda: (0, 0)),
            pl.BlockSpec((_D2, _D2), lambda: (0, 0)),
        ],
        out_specs=[
            pl.BlockSpec(memory_space=pltpu.SMEM),
            pl.BlockSpec(memory_space=pltpu.SMEM),
        ],
        out_shape=[
            jax.ShapeDtypeStruct((1, 1), jnp.float32),
            jax.ShapeDtypeStruct((1, 1), jnp.float32),
        ],
    )(head, rel, tail, svec, mom)


def kernel(anchor, rel_ind, ans_ind, entity_emb, relation_emb):
    head, rel, tail = _make_gather3()(
        entity_emb, relation_emb,
        anchor.astype(jnp.int32), rel_ind.astype(jnp.int32),
        ans_ind.astype(jnp.int32))
    svec, mom = _moments(entity_emb)
    res_fast, bnd = _assemble(head, rel, tail, svec, mom)
    return lax.cond(
        bnd[0, 0] < 0.01,
        lambda: res_fast[0, 0],
        lambda: _score(head, rel, tail, entity_emb)[0, 0],
    )


# speedup vs baseline: 1.0243x; 1.0243x over previous
import functools

import jax
import jax.numpy as jnp
from jax import lax
from jax.experimental import pallas as pl
from jax.experimental.pallas import tpu as pltpu
from jax.experimental.pallas import tpu_sc as plsc

_N_ENTITY = 100000
_N_RELATION = 500
_D = 32
_D2 = 64
_B = 1024
_REG = 0.05
_T = 2000
_NT = _N_ENTITY // _T
_TM = 25000
_TM_STEPS = _N_ENTITY // _TM

_NC, _NS = 2, 16
_NW = _NC * _NS
_BPW = _B // _NW


@functools.lru_cache(maxsize=1)
def _make_gather3():
    mesh = plsc.VectorSubcoreMesh(core_axis_name="c", subcore_axis_name="s",
                                  num_cores=_NC, num_subcores=_NS)

    @functools.partial(
        pl.kernel,
        mesh=mesh,
        out_type=[jax.ShapeDtypeStruct((_B, _D2), jnp.float32)] * 3,
        scratch_types=[
            [pltpu.VMEM((_BPW,), jnp.int32)] * 3,
            [pltpu.VMEM((_BPW, _D2), jnp.float32)] * 3,
            pltpu.SemaphoreType.DMA,
        ],
    )
    def _gather3(entity_hbm, relation_hbm, anchor_hbm, relind_hbm, ans_hbm,
                 head_out, rel_out, tail_out, idx_vs, rows_vs, sem):
        wid = lax.axis_index("s") * _NC + lax.axis_index("c")
        base = wid * _BPW
        triples = (
            (anchor_hbm, entity_hbm, head_out),
            (relind_hbm, relation_hbm, rel_out),
            (ans_hbm, entity_hbm, tail_out),
        )
        for t, (idx_hbm, table, out) in enumerate(triples):
            pltpu.sync_copy(idx_hbm.at[pl.ds(base, _BPW)], idx_vs[t])
        descs = []
        for t, (idx_hbm, table, out) in enumerate(triples):
            for c in range(_BPW // 16):
                chunk = idx_vs[t][pl.ds(c * 16, 16)]
                for k in range(16):
                    j = c * 16 + k
                    descs.append(pltpu.async_copy(
                        table.at[pl.ds(chunk[k], 1)],
                        rows_vs[t].at[pl.ds(j, 1)], sem))
        for d in descs:
            d.wait()
        for t, (idx_hbm, table, out) in enumerate(triples):
            pltpu.sync_copy(rows_vs[t], out.at[pl.ds(base, _BPW)])

    return _gather3


def _score_body(head_ref, rel_ref, tail_ref, e_ref, out_ref,
                q_ref, m_ref, s_ref, g_ref):
    i = pl.program_id(0)

    @pl.when(i == 0)
    def _init():
        head = head_ref[...]
        rel = rel_ref[...]
        tail = tail_ref[...]
        lhs0, lhs1 = head[:, :_D], head[:, _D:]
        rel0, rel1 = rel[:, :_D], rel[:, _D:]
        q0 = lhs0 * rel0 - lhs1 * rel1
        q1 = lhs0 * rel1 + lhs1 * rel0
        q_ref[...] = jnp.concatenate([q0, q1], axis=1)
        g_ref[...] = jnp.sum(q0 * tail[:, :_D] + q1 * tail[:, _D:],
                             axis=1, keepdims=True)
        m_ref[...] = jnp.full((_B, 1), -jnp.inf, dtype=jnp.float32)
        s_ref[...] = jnp.zeros((_B, 1), dtype=jnp.float32)

    q = q_ref[...]
    e = e_ref[...]
    p = lax.dot_general(q, e, (((1,), (1,)), ((), ())),
                        preferred_element_type=jnp.float32)
    m_old = m_ref[...]
    m_new = jnp.maximum(m_old, jnp.max(p, axis=1, keepdims=True))
    s_ref[...] = (s_ref[...] * jnp.exp(m_old - m_new)
                  + jnp.sum(jnp.exp(p - m_new), axis=1, keepdims=True))
    m_ref[...] = m_new

    @pl.when(i == pl.num_programs(0) - 1)
    def _fin():
        lse = m_ref[...] + jnp.log(s_ref[...])
        l_fit = jnp.mean(lse - g_ref[...])
        head = head_ref[...]
        rel = rel_ref[...]
        tail = tail_ref[...]

        def n3(x):
            a = x[:, :_D] ** 2 + x[:, _D:] ** 2
            return jnp.sum(a * jnp.sqrt(a))

        l_reg = _REG * (n3(head) + n3(rel) + n3(tail)) / _B
        out_ref[0, 0] = l_fit + l_reg


def _score(head, rel, tail, entity_emb):
    return pl.pallas_call(
        _score_body,
        grid=(_NT,),
        in_specs=[
            pl.BlockSpec((_B, _D2), lambda i: (0, 0)),
            pl.BlockSpec((_B, _D2), lambda i: (0, 0)),
            pl.BlockSpec((_B, _D2), lambda i: (0, 0)),
            pl.BlockSpec((_T, _D2), lambda i: (i, 0)),
        ],
        out_specs=pl.BlockSpec(memory_space=pltpu.SMEM),
        out_shape=jax.ShapeDtypeStruct((1, 1), jnp.float32),
        scratch_shapes=[
            pltpu.VMEM((_B, _D2), jnp.float32),
            pltpu.VMEM((_B, 1), jnp.float32),
            pltpu.VMEM((_B, 1), jnp.float32),
            pltpu.VMEM((_B, 1), jnp.float32),
        ],
        compiler_params=pltpu.CompilerParams(
            dimension_semantics=("arbitrary",)),
    )(head, rel, tail, entity_emb)


def _moments_body(e_ref, s_ref, mom_ref):
    i = pl.program_id(0)

    @pl.when(i == 0)
    def _init():
        s_ref[...] = jnp.zeros((1, _D2), dtype=jnp.float32)
        mom_ref[...] = jnp.zeros((_D2, _D2), dtype=jnp.float32)

    e = e_ref[...]
    mom_ref[...] += lax.dot_general(e, e, (((0,), (0,)), ((), ())),
                                    preferred_element_type=jnp.float32)
    ones = jnp.ones((1, _TM), dtype=jnp.float32)
    s_ref[...] += lax.dot_general(ones, e, (((1,), (0,)), ((), ())),
                                  preferred_element_type=jnp.float32)


def _moments(entity_emb):
    return pl.pallas_call(
        _moments_body,
        grid=(_TM_STEPS,),
        in_specs=[
            pl.BlockSpec((_TM, _D2), lambda i: (i, 0)),
        ],
        out_specs=[
            pl.BlockSpec((1, _D2), lambda i: (0, 0)),
            pl.BlockSpec((_D2, _D2), lambda i: (0, 0)),
        ],
        out_shape=[
            jax.ShapeDtypeStruct((1, _D2), jnp.float32),
            jax.ShapeDtypeStruct((_D2, _D2), jnp.float32),
        ],
        compiler_params=pltpu.CompilerParams(
            dimension_semantics=("arbitrary",)),
    )(entity_emb)


def _assemble_body(head_ref, rel_ref, tail_ref, s_ref, mom_ref,
                   out_ref, bnd_ref):
    head = head_ref[...]
    rel = rel_ref[...]
    tail = tail_ref[...]
    lhs0, lhs1 = head[:, :_D], head[:, _D:]
    rel0, rel1 = rel[:, :_D], rel[:, _D:]
    q0 = lhs0 * rel0 - lhs1 * rel1
    q1 = lhs0 * rel1 + lhs1 * rel0
    q = jnp.concatenate([q0, q1], axis=1)
    s1 = jnp.sum(q * s_ref[...], axis=1, keepdims=True)
    qm = lax.dot_general(q, mom_ref[...], (((1,), (0,)), ((), ())),
                         preferred_element_type=jnp.float32)
    s2 = jnp.sum(qm * q, axis=1, keepdims=True)
    lse = jnp.log(jnp.float32(_N_ENTITY) + s1 + 0.5 * s2)
    gold = jnp.sum(q * tail, axis=1, keepdims=True)
    l_fit = jnp.mean(lse - gold)

    def n3(x):
        a = x[:, :_D] ** 2 + x[:, _D:] ** 2
        return jnp.sum(a * jnp.sqrt(a))

    l_reg = _REG * (n3(head) + n3(rel) + n3(tail)) / _B
    out_ref[0, 0] = l_fit + l_reg
    mom = mom_ref[...]
    row = lax.broadcasted_iota(jnp.int32, (_D2, _D2), 0)
    col = lax.broadcasted_iota(jnp.int32, (_D2, _D2), 1)
    tr = jnp.sum(jnp.where(row == col, mom, 0.0))
    maxq2 = jnp.max(jnp.sum(q * q, axis=1, keepdims=True))
    bnd_ref[0, 0] = jnp.sqrt(maxq2 * tr)


def _assemble(head, rel, tail, svec, mom):
    return pl.pallas_call(
        _assemble_body,
        in_specs=[
            pl.BlockSpec((_B, _D2), lambda: (0, 0)),
            pl.BlockSpec((_B, _D2), lambda: (0, 0)),
            pl.BlockSpec((_B, _D2), lambda: (0, 0)),
            pl.BlockSpec((1, _D2), lambda: (0, 0)),
            pl.BlockSpec((_D2, _D2), lambda: (0, 0)),
        ],
        out_specs=[
            pl.BlockSpec(memory_space=pltpu.SMEM),
            pl.BlockSpec(memory_space=pltpu.SMEM),
        ],
        out_shape=[
            jax.ShapeDtypeStruct((1, 1), jnp.float32),
            jax.ShapeDtypeStruct((1, 1), jnp.float32),
        ],
    )(head, rel, tail, svec, mom)


def kernel(anchor, rel_ind, ans_ind, entity_emb, relation_emb):
    head, rel, tail = _make_gather3()(
        entity_emb, relation_emb,
        anchor.astype(jnp.int32), rel_ind.astype(jnp.int32),
        ans_ind.astype(jnp.int32))
    svec, mom = _moments(entity_emb)
    res_fast, bnd = _assemble(head, rel, tail, svec, mom)
    return lax.cond(
        bnd[0, 0] < 0.01,
        lambda: res_fast[0, 0],
        lambda: _score(head, rel, tail, entity_emb)[0, 0],
    )
